# baseline (speedup 1.0000x reference)
import functools

import jax
import jax.numpy as jnp
from jax import lax
from jax.experimental import pallas as pl
from jax.experimental.pallas import tpu as pltpu
from jax.experimental.pallas import tpu_sc as plsc

B = 16384
D = 64
DW = D // 2
NC = 2
NS = 16
NW = NC * NS
BPW = B // NW
GROUPS = BPW // 16


def _neg_sqrt(s):
    xs = jnp.maximum(s, jnp.float32(1e-37))
    i = lax.bitcast_convert_type(xs, jnp.int32)
    r = lax.bitcast_convert_type(jnp.int32(0x5F3759DF) - (i >> 1), jnp.float32)
    for _ in range(3):
        r = r * (jnp.float32(1.5) - jnp.float32(0.5) * xs * r * r)
    return -(xs * r)


def _unpack2(v_i32):
    lo = lax.bitcast_convert_type(lax.shift_left(v_i32, 16), jnp.float32)
    hi = lax.bitcast_convert_type(
        lax.bitwise_and(v_i32, jnp.int32(-65536)), jnp.float32)
    return lo, hi


def _body(x_hbm, y_hbm, tab_hbm, out_hbm,
          idx_x, idx_y, rows_x, rows_y, out_v, sem_x, sem_y):
    c = lax.axis_index("c")
    s = lax.axis_index("s")
    wid = s * NC + c
    base = wid * BPW

    pltpu.sync_copy(x_hbm.at[pl.ds(base, BPW)], idx_x)
    pltpu.sync_copy(y_hbm.at[pl.ds(base, BPW)], idx_y)
    cp_x = pltpu.async_copy(tab_hbm.at[idx_x], rows_x, sem_x)
    cp_y = pltpu.async_copy(tab_hbm.at[idx_y], rows_y, sem_y)
    cp_x.wait()
    cp_y.wait()

    lane = lax.iota(jnp.int32, 16)

    def g_body(g, carry):
        row = g * 16 + lane
        acc = jnp.zeros((16,), jnp.float32)
        for w in range(DW):
            col = jnp.full((16,), w, jnp.int32)
            xv = plsc.load_gather(rows_x, [row, col])
            yv = plsc.load_gather(rows_y, [row, col])
            xlo, xhi = _unpack2(xv)
            ylo, yhi = _unpack2(yv)
            t0 = xlo - ylo
            t1 = xhi - yhi
            acc = acc + t0 * t0 + t1 * t1
        out_v[pl.ds(g * 16, 16)] = _neg_sqrt(acc)
        return carry

    lax.fori_loop(0, GROUPS, g_body, 0)
    pltpu.sync_copy(out_v, out_hbm.at[pl.ds(base, BPW)])


@jax.jit
def kernel(x, y, entity_embeddings):
    tab16 = entity_embeddings.astype(jnp.bfloat16)
    tabi = lax.bitcast_convert_type(
        tab16.reshape(1_000_000, DW, 2), jnp.int32)
    mesh = plsc.VectorSubcoreMesh(core_axis_name="c", subcore_axis_name="s")
    run = functools.partial(
        pl.kernel,
        out_type=jax.ShapeDtypeStruct((B,), jnp.float32),
        mesh=mesh,
        compiler_params=pltpu.CompilerParams(
            use_tc_tiling_on_sc=False, needs_layout_passes=False),
        scratch_types=[
            pltpu.VMEM((BPW,), jnp.int32),
            pltpu.VMEM((BPW,), jnp.int32),
            pltpu.VMEM((BPW, DW), jnp.int32),
            pltpu.VMEM((BPW, DW), jnp.int32),
            pltpu.VMEM((BPW,), jnp.float32),
            pltpu.SemaphoreType.DMA,
            pltpu.SemaphoreType.DMA,
        ],
    )(_body)
    return run(x.astype(jnp.int32), y.astype(jnp.int32), tabi)


# speedup vs baseline: 5.6965x; 5.6965x over previous
import functools

import jax
import jax.numpy as jnp
from jax import lax
from jax.experimental import pallas as pl
from jax.experimental.pallas import tpu as pltpu
from jax.experimental.pallas import tpu_sc as plsc

B = 16384
D = 64
V = 1_000_000
QB = 249856
CH = 2048
NSTEP = 123
S = NSTEP * CH

NC = 2
NS = 16
NW = NC * NS
BPW = B // NW
HB = 256
GPH = HB // 16


def _pack_bf16(wbits):
    t = (wbits + jnp.int32(0x7FFF) + ((wbits >> 16) & jnp.int32(1))) >> 16
    t = t & jnp.int32(0xFFFF)
    return t[0:32, :] | (t[32:64, :] << 16)


def _tc_body(t0, t1, t2, t3, out_ref):
    parts = []
    for ref in (t0, t1, t2, t3):
        w = lax.bitcast_convert_type(ref[...], jnp.int32)
        parts.append(_pack_bf16(w).T)
    out_ref[...] = jnp.concatenate(parts, axis=1)


def _relayout(table):
    tt = table.T
    specs = [
        pl.BlockSpec((64, CH), lambda i, q=q: (0, 122 * q + i))
        for q in range(4)
    ]
    return pl.pallas_call(
        _tc_body,
        grid=(NSTEP,),
        in_specs=specs,
        out_specs=pl.BlockSpec((CH, 128), lambda i: (i, 0)),
        out_shape=jax.ShapeDtypeStruct((S, 128), jnp.int32),
    )(tt, tt, tt, tt)


def _neg_sqrt(s):
    xs = jnp.maximum(s, jnp.float32(1e-37))
    i = lax.bitcast_convert_type(xs, jnp.int32)
    r = lax.bitcast_convert_type(jnp.int32(0x5F3759DF) - (i >> 1), jnp.float32)
    for _ in range(3):
        r = r * (jnp.float32(1.5) - jnp.float32(0.5) * xs * r * r)
    return -(xs * r)


def _quarter(v):
    one = jnp.int32(1)
    zero = jnp.int32(0)
    q = jnp.where(v >= jnp.int32(QB), one, zero)
    q = q + jnp.where(v >= jnp.int32(2 * QB), one, zero)
    q = q + jnp.where(v >= jnp.int32(3 * QB), one, zero)
    return q


def _unpack2(v_i32):
    lo = lax.bitcast_convert_type(lax.shift_left(v_i32, 16), jnp.float32)
    hi = lax.bitcast_convert_type(
        lax.bitwise_and(v_i32, jnp.int32(-65536)), jnp.float32)
    return lo, hi


def _sc_body(x_hbm, y_hbm, tab_hbm, out_hbm,
             idx_xv, idx_yv, mod_x, mod_y, dst_x, dst_y, out_v,
             sem_x, sem_y):
    c = lax.axis_index("c")
    s = lax.axis_index("s")
    wid = s * NC + c
    base = wid * BPW

    pltpu.sync_copy(x_hbm.at[pl.ds(base, BPW)], idx_xv)
    pltpu.sync_copy(y_hbm.at[pl.ds(base, BPW)], idx_yv)

    lane = lax.iota(jnp.int32, 16)

    def half(h, carry):
        for b in range(GPH):
            vx = idx_xv[pl.ds(h * HB + b * 16, 16)]
            vy = idx_yv[pl.ds(h * HB + b * 16, 16)]
            mod_x[pl.ds(b * 16, 16)] = vx - _quarter(vx) * jnp.int32(QB)
            mod_y[pl.ds(b * 16, 16)] = vy - _quarter(vy) * jnp.int32(QB)
        cp_x = pltpu.async_copy(tab_hbm.at[mod_x], dst_x, sem_x)
        cp_y = pltpu.async_copy(tab_hbm.at[mod_y], dst_y, sem_y)
        cp_x.wait()
        cp_y.wait()
        for g in range(GPH):
            slot = g * 16 + lane
            ivx = idx_xv[pl.ds(h * HB + g * 16, 16)]
            ivy = idx_yv[pl.ds(h * HB + g * 16, 16)]
            cbx = _quarter(ivx) * jnp.int32(32)
            cby = _quarter(ivy) * jnp.int32(32)
            acc = jnp.zeros((16,), jnp.float32)
            for w in range(32):
                xv = plsc.load_gather(dst_x, [slot, cbx + w])
                yv = plsc.load_gather(dst_y, [slot, cby + w])
                xlo, xhi = _unpack2(xv)
                ylo, yhi = _unpack2(yv)
                t0 = xlo - ylo
                t1 = xhi - yhi
                acc = acc + t0 * t0 + t1 * t1
            out_v[pl.ds(h * HB + g * 16, 16)] = _neg_sqrt(acc)
        return carry

    lax.fori_loop(0, BPW // HB, half, 0)
    pltpu.sync_copy(out_v, out_hbm.at[pl.ds(base, BPW)])


@jax.jit
def kernel(x, y, entity_embeddings):
    packed = _relayout(entity_embeddings)
    mesh = plsc.VectorSubcoreMesh(core_axis_name="c", subcore_axis_name="s")
    run = functools.partial(
        pl.kernel,
        out_type=jax.ShapeDtypeStruct((B,), jnp.float32),
        mesh=mesh,
        compiler_params=pltpu.CompilerParams(
            use_tc_tiling_on_sc=False, needs_layout_passes=False),
        scratch_types=[
            pltpu.VMEM((BPW,), jnp.int32),
            pltpu.VMEM((BPW,), jnp.int32),
            pltpu.VMEM((HB,), jnp.int32),
            pltpu.VMEM((HB,), jnp.int32),
            pltpu.VMEM((HB, 128), jnp.int32),
            pltpu.VMEM((HB, 128), jnp.int32),
            pltpu.VMEM((BPW,), jnp.float32),
            pltpu.SemaphoreType.DMA,
            pltpu.SemaphoreType.DMA,
        ],
    )(_sc_body)
    return run(x.astype(jnp.int32), y.astype(jnp.int32), packed)


# speedup vs baseline: 5.7156x; 1.0033x over previous
import functools

import jax
import jax.numpy as jnp
from jax import lax
from jax.experimental import pallas as pl
from jax.experimental.pallas import tpu as pltpu
from jax.experimental.pallas import tpu_sc as plsc

B = 16384
D = 64
V = 1_000_000
QB = 249856
CH = 2048
NSTEP = 123
S = NSTEP * CH

NC = 2
NS = 16
NW = NC * NS
BPW = B // NW
HB = 256
GPH = HB // 16


def _pack_bf16(wbits):
    lo = lax.shift_right_logical(wbits[0:32, :], 16)
    hi = wbits[32:64, :] & jnp.int32(-65536)
    return lo | hi


def _tc_body(t0, t1, t2, t3, out_ref):
    parts = []
    for ref in (t0, t1, t2, t3):
        w = lax.bitcast_convert_type(ref[...], jnp.int32)
        parts.append(_pack_bf16(w).T)
    out_ref[...] = jnp.concatenate(parts, axis=1)


def _relayout(table):
    tt = table.T
    specs = [
        pl.BlockSpec((64, CH), lambda i, q=q: (0, 122 * q + i))
        for q in range(4)
    ]
    return pl.pallas_call(
        _tc_body,
        grid=(NSTEP,),
        in_specs=specs,
        out_specs=pl.BlockSpec((CH, 128), lambda i: (i, 0)),
        out_shape=jax.ShapeDtypeStruct((S, 128), jnp.int32),
    )(tt, tt, tt, tt)


def _neg_sqrt(s):
    xs = jnp.maximum(s, jnp.float32(1e-37))
    i = lax.bitcast_convert_type(xs, jnp.int32)
    r = lax.bitcast_convert_type(jnp.int32(0x5F3759DF) - (i >> 1), jnp.float32)
    for _ in range(3):
        r = r * (jnp.float32(1.5) - jnp.float32(0.5) * xs * r * r)
    return -(xs * r)


def _quarter(v):
    one = jnp.int32(1)
    zero = jnp.int32(0)
    q = jnp.where(v >= jnp.int32(QB), one, zero)
    q = q + jnp.where(v >= jnp.int32(2 * QB), one, zero)
    q = q + jnp.where(v >= jnp.int32(3 * QB), one, zero)
    return q


def _unpack2(v_i32):
    lo = lax.bitcast_convert_type(lax.shift_left(v_i32, 16), jnp.float32)
    hi = lax.bitcast_convert_type(
        lax.bitwise_and(v_i32, jnp.int32(-65536)), jnp.float32)
    return lo, hi


def _sc_body(x_hbm, y_hbm, tab_hbm, out_hbm,
             idx_xv, idx_yv, mod_x, mod_y, dst_x, dst_y, out_v,
             sem_x, sem_y):
    c = lax.axis_index("c")
    s = lax.axis_index("s")
    wid = s * NC + c
    base = wid * BPW

    pltpu.sync_copy(x_hbm.at[pl.ds(base, BPW)], idx_xv)
    pltpu.sync_copy(y_hbm.at[pl.ds(base, BPW)], idx_yv)

    lane = lax.iota(jnp.int32, 16)

    def half(h, carry):
        for b in range(GPH):
            vx = idx_xv[pl.ds(h * HB + b * 16, 16)]
            vy = idx_yv[pl.ds(h * HB + b * 16, 16)]
            mod_x[pl.ds(b * 16, 16)] = vx - _quarter(vx) * jnp.int32(QB)
            mod_y[pl.ds(b * 16, 16)] = vy - _quarter(vy) * jnp.int32(QB)
        cp_x = pltpu.async_copy(tab_hbm.at[mod_x], dst_x, sem_x)
        cp_y = pltpu.async_copy(tab_hbm.at[mod_y], dst_y, sem_y)
        cp_x.wait()
        cp_y.wait()
        for g in range(GPH):
            slot = g * 16 + lane
            ivx = idx_xv[pl.ds(h * HB + g * 16, 16)]
            ivy = idx_yv[pl.ds(h * HB + g * 16, 16)]
            cbx = _quarter(ivx) * jnp.int32(32)
            cby = _quarter(ivy) * jnp.int32(32)
            acc = jnp.zeros((16,), jnp.float32)
            for w in range(32):
                xv = plsc.load_gather(dst_x, [slot, cbx + w])
                yv = plsc.load_gather(dst_y, [slot, cby + w])
                xlo, xhi = _unpack2(xv)
                ylo, yhi = _unpack2(yv)
                t0 = xlo - ylo
                t1 = xhi - yhi
                acc = acc + t0 * t0 + t1 * t1
            out_v[pl.ds(h * HB + g * 16, 16)] = _neg_sqrt(acc)
        return carry

    lax.fori_loop(0, BPW // HB, half, 0)
    pltpu.sync_copy(out_v, out_hbm.at[pl.ds(base, BPW)])


@jax.jit
def kernel(x, y, entity_embeddings):
    packed = _relayout(entity_embeddings)
    mesh = plsc.VectorSubcoreMesh(core_axis_name="c", subcore_axis_name="s")
    run = functools.partial(
        pl.kernel,
        out_type=jax.ShapeDtypeStruct((B,), jnp.float32),
        mesh=mesh,
        compiler_params=pltpu.CompilerParams(
            use_tc_tiling_on_sc=False, needs_layout_passes=False),
        scratch_types=[
            pltpu.VMEM((BPW,), jnp.int32),
            pltpu.VMEM((BPW,), jnp.int32),
            pltpu.VMEM((HB,), jnp.int32),
            pltpu.VMEM((HB,), jnp.int32),
            pltpu.VMEM((HB, 128), jnp.int32),
            pltpu.VMEM((HB, 128), jnp.int32),
            pltpu.VMEM((BPW,), jnp.float32),
            pltpu.SemaphoreType.DMA,
            pltpu.SemaphoreType.DMA,
        ],
    )(_sc_body)
    return run(x.astype(jnp.int32), y.astype(jnp.int32), packed)
